# baseline (speedup 1.0000x reference)
import functools

import jax
import jax.numpy as jnp
from jax import lax
from jax.experimental import pallas as pl
from jax.experimental.pallas import tpu as pltpu
from jax.experimental.pallas import tpu_sc as plsc

BATCH = 4096
DIM = 128

_NC = 2
_NS = 16
_NW = _NC * _NS
_B_PER_W = BATCH // _NW

_mesh = plsc.VectorSubcoreMesh(core_axis_name="c", subcore_axis_name="s")


_C0 = _B_PER_W // 2
_C1 = _B_PER_W - _C0


@functools.partial(
    pl.kernel,
    mesh=_mesh,
    out_type=jax.ShapeDtypeStruct((BATCH, DIM), jnp.float32),
    scratch_types=[
        pltpu.VMEM((_B_PER_W,), jnp.int32),
        pltpu.VMEM((_C0, DIM), jnp.float32),
        pltpu.VMEM((_C1, DIM), jnp.float32),
        pltpu.SemaphoreType.DMA,
        pltpu.SemaphoreType.DMA,
        pltpu.SemaphoreType.DMA,
        pltpu.SemaphoreType.DMA,
    ],
)
def _sc_gather(idx_hbm, table_hbm, out_hbm,
               idx_v, rows0, rows1, sg0, sg1, ss0, ss1):
    wid = lax.axis_index("s") * _NC + lax.axis_index("c")
    base = wid * _B_PER_W
    pltpu.sync_copy(idx_hbm.at[pl.ds(base, _B_PER_W)], idx_v)
    g0 = pltpu.async_copy(table_hbm.at[idx_v.at[pl.ds(0, _C0)]], rows0, sg0)
    g1 = pltpu.async_copy(table_hbm.at[idx_v.at[pl.ds(_C0, _C1)]], rows1, sg1)
    g0.wait()
    s0 = pltpu.async_copy(rows0, out_hbm.at[pl.ds(base, _C0)], ss0)
    g1.wait()
    s1 = pltpu.async_copy(rows1, out_hbm.at[pl.ds(base + _C0, _C1)], ss1)
    s0.wait()
    s1.wait()


def kernel(organism_index, embed_weight):
    idx = organism_index.astype(jnp.int32)
    return _sc_gather(idx, embed_weight)
